# baseline (speedup 1.0000x reference)
import functools

import jax
import jax.numpy as jnp
from jax import lax
from jax.experimental import pallas as pl
from jax.experimental.pallas import tpu as pltpu
from jax.experimental.pallas import tpu_sc as plsc

D_MODEL = 64
SCALE = 8.0
CHUNK = 128
NBUF = 4


@functools.lru_cache(maxsize=None)
def _make_kernel(B: int):
    info = plsc.get_sparse_core_info()
    nc, ns = info.num_cores, info.num_subcores
    nw = nc * ns
    b_per_w = B // nw
    n_chunks = b_per_w // CHUNK
    n_outer = n_chunks // NBUF
    orows_per_w = b_per_w // 2
    assert b_per_w * nw == B and n_outer * NBUF == n_chunks

    mesh = plsc.VectorSubcoreMesh(core_axis_name="c", subcore_axis_name="s")

    @functools.partial(
        pl.kernel,
        mesh=mesh,
        out_type=jax.ShapeDtypeStruct((B // 2, 128), jnp.float32),
        compiler_params=pltpu.CompilerParams(use_tc_tiling_on_sc=True),
        scratch_types=(
            [pltpu.VMEM((b_per_w + 16,), jnp.int32)]
            + [pltpu.VMEM((CHUNK,), jnp.int32) for _ in range(NBUF)]
            + [pltpu.VMEM((CHUNK, 128), jnp.float32) for _ in range(NBUF)]
            + [pltpu.VMEM((CHUNK // 2, 128), jnp.float32) for _ in range(NBUF)]
            + [pltpu.SemaphoreType.DMA for _ in range(NBUF)]
        ),
    )
    def emb_kernel(x_hbm, lut_hbm, out_hbm, idx_v, *rest):
        ibufs = rest[:NBUF]
        gbufs = rest[NBUF:2 * NBUF]
        obufs = rest[2 * NBUF:3 * NBUF]
        sems = rest[3 * NBUF:]
        wid = lax.axis_index("s") * nc + lax.axis_index("c")
        base = wid * b_per_w

        pltpu.sync_copy(x_hbm.at[pl.ds(base, b_per_w)], idx_v.at[pl.ds(0, b_per_w)])

        def start_gather(g, b):
            off = pl.multiple_of(g * CHUNK, CHUNK)
            for c in range(CHUNK // 16):
                sl = pl.ds(c * 16, 16)
                ibufs[b][sl] = lax.shift_right_logical(
                    idx_v[pl.ds(off + c * 16, 16)], 1
                )
            pltpu.make_async_copy(
                lut_hbm.at[ibufs[b]], gbufs[b], sems[b]
            ).start()

        def wait_gather(b):
            pltpu.make_async_copy(
                lut_hbm.at[ibufs[b]], gbufs[b], sems[b]
            ).wait()

        for b in range(NBUF):
            start_gather(b, b)

        def outer(o, carry):
            g0 = o * NBUF
            for b in range(NBUF):
                g = g0 + b
                wait_gather(b)

                gbuf, obuf = gbufs[b], obufs[b]
                ioff = pl.multiple_of(g * CHUNK, CHUNK)

                def select_rows(r, c2, gbuf=gbuf, obuf=obuf, ioff=ioff):
                    pair = idx_v[pl.ds(ioff + 2 * r, 16)]
                    p0 = (pair[0] & 1) * 64
                    p1 = (pair[1] & 1) * 64
                    for c in range(4):
                        obuf[r, pl.ds(c * 16, 16)] = (
                            gbuf[2 * r, pl.ds(p0 + c * 16, 16)] * SCALE
                        )
                        obuf[r, pl.ds(64 + c * 16, 16)] = (
                            gbuf[2 * r + 1, pl.ds(p1 + c * 16, 16)] * SCALE
                        )
                    return c2

                lax.fori_loop(0, CHUNK // 2, select_rows, 0, unroll=2)

                @pl.when(g + NBUF < n_chunks)
                def _():
                    start_gather(g + NBUF, b)

                orow = pl.multiple_of(
                    wid * orows_per_w + g * (CHUNK // 2), CHUNK // 2
                )
                pltpu.sync_copy(obuf, out_hbm.at[pl.ds(orow, CHUNK // 2)])
            return carry

        lax.fori_loop(0, n_outer, outer, 0)

    return emb_kernel


def kernel(x, lut):
    B = x.shape[0] * x.shape[1]
    xf = x.reshape(B).astype(jnp.int32)
    lut128 = lut.reshape(lut.shape[0] // 2, 128)
    out = _make_kernel(B)(xf, lut128)
    return out.reshape(x.shape[0], x.shape[1], D_MODEL)


# speedup vs baseline: 1.1587x; 1.1587x over previous
import functools

import jax
import jax.numpy as jnp
from jax import lax
from jax.experimental import pallas as pl
from jax.experimental.pallas import tpu as pltpu
from jax.experimental.pallas import tpu_sc as plsc

D_MODEL = 64
SCALE = 8.0
CHUNK = 128
NBUF = 4


@functools.lru_cache(maxsize=None)
def _make_kernel(B: int, V: int):
    info = plsc.get_sparse_core_info()
    nc, ns = info.num_cores, info.num_subcores
    nw = nc * ns
    b_per_w = B // nw
    n_chunks = b_per_w // CHUNK
    n_outer = n_chunks // NBUF
    orows_per_w = b_per_w // 2
    assert b_per_w * nw == B and n_outer * NBUF == n_chunks

    mesh = plsc.VectorSubcoreMesh(core_axis_name="c", subcore_axis_name="s")

    @functools.partial(
        pl.kernel,
        mesh=mesh,
        out_type=jax.ShapeDtypeStruct((B // 2, 128), jnp.float32),
        compiler_params=pltpu.CompilerParams(use_tc_tiling_on_sc=True),
        scratch_types=(
            [pltpu.VMEM((b_per_w,), jnp.int32)]
            + [pltpu.VMEM((CHUNK, 128), jnp.float32) for _ in range(NBUF)]
            + [pltpu.VMEM((CHUNK // 2, 128), jnp.float32) for _ in range(NBUF)]
            + [pltpu.SemaphoreType.DMA for _ in range(NBUF)]
        ),
    )
    def emb_kernel(x_hbm, lut_hbm, out_hbm, idx_v, *rest):
        gbufs = rest[:NBUF]
        obufs = rest[NBUF:2 * NBUF]
        sems = rest[2 * NBUF:]
        wid = lax.axis_index("s") * nc + lax.axis_index("c")
        base = wid * b_per_w

        pltpu.sync_copy(x_hbm.at[pl.ds(base, b_per_w)], idx_v)

        def start_gather(g, b):
            idx_slice = idx_v.at[pl.ds(pl.multiple_of(g * CHUNK, CHUNK), CHUNK)]
            pltpu.async_copy(lut_hbm.at[idx_slice], gbufs[b], sems[b])

        def wait_gather(g, b):
            idx_slice = idx_v.at[pl.ds(pl.multiple_of(g * CHUNK, CHUNK), CHUNK)]
            pltpu.make_async_copy(lut_hbm.at[idx_slice], gbufs[b], sems[b]).wait()

        for b in range(NBUF):
            start_gather(b, b)

        def outer(o, carry):
            g0 = o * NBUF
            for b in range(NBUF):
                g = g0 + b
                wait_gather(g, b)

                gbuf, obuf = gbufs[b], obufs[b]

                def pack_rows(r, c2, gbuf=gbuf, obuf=obuf):
                    for c in range(4):
                        sl = pl.ds(c * 16, 16)
                        obuf[r, sl] = gbuf[2 * r, sl] * SCALE
                        obuf[r, pl.ds(64 + c * 16, 16)] = (
                            gbuf[2 * r + 1, sl] * SCALE
                        )
                    return c2

                lax.fori_loop(0, CHUNK // 2, pack_rows, 0, unroll=4)

                @pl.when(g + NBUF < n_chunks)
                def _():
                    start_gather(g + NBUF, b)

                orow = pl.multiple_of(
                    wid * orows_per_w + g * (CHUNK // 2), CHUNK // 2
                )
                pltpu.sync_copy(obuf, out_hbm.at[pl.ds(orow, CHUNK // 2)])
            return carry

        lax.fori_loop(0, n_outer, outer, 0)

    return emb_kernel


def kernel(x, lut):
    B = x.shape[0] * x.shape[1]
    xf = x.reshape(B).astype(jnp.int32)
    lutp = jnp.pad(lut, ((0, 0), (0, 128 - lut.shape[1])))
    out = _make_kernel(B, lut.shape[0])(xf, lutp)
    return out.reshape(x.shape[0], x.shape[1], D_MODEL)
